# Initial kernel scaffold; baseline (speedup 1.0000x reference)
import jax
import jax.numpy as jnp
from jax.experimental import pallas as pl


def kernel(features, labels):
    raise NotImplementedError("write your pallas kernel here")


# speedup vs baseline: 14.7045x; 14.7045x over previous
import functools

import jax
import jax.numpy as jnp
from jax import lax
from jax.experimental import pallas as pl
from jax.experimental.pallas import tpu as pltpu

MARGIN = 0.4
NUM_CLASSES = 64
BATCH = 4096
DIM = 256
BLK = 512
GRID = BATCH // BLK


def _body(feats_ref, labels_ref, out_ref, acc_ref, cnt_ref):
    i = pl.program_id(0)
    feats = feats_ref[...]
    labels = labels_ref[0, 0, :]
    nrm = jnp.sqrt(jnp.sum(feats * feats, axis=1, keepdims=True))
    fn = feats / jnp.maximum(nrm, 1e-12)
    onehot = (
        labels[None, :] == lax.broadcasted_iota(jnp.int32, (NUM_CLASSES, BLK), 0)
    ).astype(jnp.float32)
    psum = jnp.dot(onehot, fn, preferred_element_type=jnp.float32)
    pcnt = jnp.sum(onehot, axis=1, keepdims=True)

    @pl.when(i == 0)
    def _():
        acc_ref[...] = psum
        cnt_ref[...] = jnp.broadcast_to(pcnt, cnt_ref.shape)

    @pl.when(i > 0)
    def _():
        acc_ref[...] += psum
        cnt_ref[...] += jnp.broadcast_to(pcnt, cnt_ref.shape)

    @pl.when(i == GRID - 1)
    def _():
        sums = acc_ref[...]
        counts = cnt_ref[:, 0:1]
        cu = sums / jnp.maximum(counts, 1.0)
        ncn = jnp.sqrt(jnp.sum(cu * cu, axis=1, keepdims=True))
        centers = cu / jnp.maximum(ncn, 1e-12)
        intra_mean = jnp.sum(sums * centers) / BATCH
        intra_loss = 1.0 - intra_mean
        csim = jnp.dot(centers, centers.T, preferred_element_type=jnp.float32)
        r = lax.broadcasted_iota(jnp.int32, (NUM_CLASSES, NUM_CLASSES), 0)
        c = lax.broadcasted_iota(jnp.int32, (NUM_CLASSES, NUM_CLASSES), 1)
        max_inter = jnp.max(jnp.where(r == c, -jnp.inf, csim))
        inter_loss = jnp.maximum(max_inter - MARGIN, 0.0)
        ratio = jnp.clip((max_inter - MARGIN) / (1.0 - MARGIN), 0.0, 1.0)
        out_ref[0, 0] = (1.0 + 2.0 * ratio) * intra_loss + 2.0 * (
            1.0 - ratio
        ) * inter_loss


@jax.jit
def kernel(features, labels):
    labels3 = labels.reshape(GRID, 1, BLK)
    out = pl.pallas_call(
        _body,
        grid=(GRID,),
        in_specs=[
            pl.BlockSpec((BLK, DIM), lambda i: (i, 0)),
            pl.BlockSpec((1, 1, BLK), lambda i: (i, 0, 0)),
        ],
        out_specs=pl.BlockSpec(memory_space=pltpu.SMEM),
        out_shape=jax.ShapeDtypeStruct((1, 1), jnp.float32),
        scratch_shapes=[
            pltpu.VMEM((NUM_CLASSES, DIM), jnp.float32),
            pltpu.VMEM((NUM_CLASSES, 128), jnp.float32),
        ],
    )(features, labels3)
    return out[0, 0]
